# baseline (speedup 1.0000x reference)
import functools

import jax
import jax.numpy as jnp
from jax import lax
from jax.experimental import pallas as pl
from jax.experimental.pallas import tpu as pltpu
from jax.experimental.pallas import tpu_sc as plsc

B = 4096
EMB = 64
T_LEN = 20
R_LEN = 200
H1 = 256
H2 = 128
NCLS = 2

NUM_CORES = 2
NUM_SUBCORES = 16
NW = NUM_CORES * NUM_SUBCORES
BPW = B // NW
NPAIR = BPW // 2

RC0 = 104
RC1 = R_LEN - RC0


def _bow_body(title_hbm, review_hbm, emb_hbm, out_hbm,
              tidx_v, ridx_v, tbuf, rbuf0, rbuf1, acc_v, sem):
    wid = lax.axis_index("s") * NUM_CORES + lax.axis_index("c")
    base = wid * BPW

    pltpu.sync_copy(title_hbm.at[pl.ds(base * T_LEN, BPW * T_LEN)], tidx_v)
    pltpu.sync_copy(review_hbm.at[pl.ds(base * R_LEN, BPW * R_LEN)], ridx_v)

    def pair_body(b2, carry):
        r0 = 2 * b2
        r1 = r0 + 1
        h0 = pltpu.async_copy(emb_hbm.at[tidx_v.at[pl.ds(b2 * (2 * T_LEN), 2 * T_LEN)]], tbuf, sem)
        h1 = pltpu.async_copy(emb_hbm.at[ridx_v.at[pl.ds(r0 * R_LEN, RC0)]], rbuf0.at[pl.ds(0, RC0)], sem)
        h2 = pltpu.async_copy(emb_hbm.at[ridx_v.at[pl.ds(r0 * R_LEN + RC0, RC1)]], rbuf0.at[pl.ds(RC0, RC1)], sem)
        h3 = pltpu.async_copy(emb_hbm.at[ridx_v.at[pl.ds(r1 * R_LEN, RC0)]], rbuf1.at[pl.ds(0, RC0)], sem)
        h4 = pltpu.async_copy(emb_hbm.at[ridx_v.at[pl.ds(r1 * R_LEN + RC0, RC1)]], rbuf1.at[pl.ds(RC0, RC1)], sem)
        h0.wait()
        h1.wait()
        h2.wait()
        h3.wait()
        h4.wait()

        zero = jnp.zeros((16,), jnp.float32)

        def tsum(t, accs):
            return tuple(a + tbuf[t, pl.ds(16 * c, 16)] for c, a in enumerate(accs))

        ta0 = lax.fori_loop(0, T_LEN, tsum, (zero,) * 4, unroll=4)
        ta1 = lax.fori_loop(T_LEN, 2 * T_LEN, tsum, (zero,) * 4, unroll=4)

        def rsum0(t, accs):
            return tuple(a + rbuf0[t, pl.ds(16 * c, 16)] for c, a in enumerate(accs))

        def rsum1(t, accs):
            return tuple(a + rbuf1[t, pl.ds(16 * c, 16)] for c, a in enumerate(accs))

        ra0 = lax.fori_loop(0, R_LEN, rsum0, (zero,) * 4, unroll=8)
        ra1 = lax.fori_loop(0, R_LEN, rsum1, (zero,) * 4, unroll=8)

        for c in range(4):
            acc_v[r0, pl.ds(16 * c, 16)] = ta0[c]
            acc_v[r1, pl.ds(16 * c, 16)] = ta1[c]
            acc_v[r0, pl.ds(EMB + 16 * c, 16)] = ra0[c]
            acc_v[r1, pl.ds(EMB + 16 * c, 16)] = ra1[c]
        return carry

    lax.fori_loop(0, NPAIR, pair_body, 0)
    pltpu.sync_copy(acc_v, out_hbm.at[pl.ds(base, BPW)])


@jax.jit
def _bow(title_flat, review_flat, emb):
    mesh = plsc.VectorSubcoreMesh(core_axis_name="c", subcore_axis_name="s")
    return pl.kernel(
        _bow_body,
        out_type=jax.ShapeDtypeStruct((B, 2 * EMB), jnp.float32),
        mesh=mesh,
        scratch_types=[
            pltpu.VMEM((BPW * T_LEN,), jnp.int32),
            pltpu.VMEM((BPW * R_LEN,), jnp.int32),
            pltpu.VMEM((2 * T_LEN, EMB), jnp.float32),
            pltpu.VMEM((R_LEN, EMB), jnp.float32),
            pltpu.VMEM((R_LEN, EMB), jnp.float32),
            pltpu.VMEM((BPW, 2 * EMB), jnp.float32),
            pltpu.SemaphoreType.DMA,
        ],
        compiler_params=pltpu.CompilerParams(use_tc_tiling_on_sc=False),
    )(title_flat, review_flat, emb)


def _mlp_body(x_ref, w1_ref, b1_ref, w2_ref, b2_ref, w3_ref, b3_ref, o_ref):
    x = x_ref[...]
    h = jnp.dot(x, w1_ref[...], preferred_element_type=jnp.float32) + b1_ref[...]
    h = jnp.maximum(h, 0.0)
    h = jnp.dot(h, w2_ref[...], preferred_element_type=jnp.float32) + b2_ref[...]
    h = jnp.maximum(h, 0.0)
    o_ref[...] = jnp.dot(h, w3_ref[...], preferred_element_type=jnp.float32) + b3_ref[...]


def _mlp(x, w1t, b1r, w2t, b2r, w3t, b3r):
    BLK = 512
    return pl.pallas_call(
        _mlp_body,
        grid=(B // BLK,),
        in_specs=[
            pl.BlockSpec((BLK, 2 * EMB), lambda i: (i, 0)),
            pl.BlockSpec((2 * EMB, H1), lambda i: (0, 0)),
            pl.BlockSpec((1, H1), lambda i: (0, 0)),
            pl.BlockSpec((H1, H2), lambda i: (0, 0)),
            pl.BlockSpec((1, H2), lambda i: (0, 0)),
            pl.BlockSpec((H2, NCLS), lambda i: (0, 0)),
            pl.BlockSpec((1, NCLS), lambda i: (0, 0)),
        ],
        out_specs=pl.BlockSpec((BLK, NCLS), lambda i: (i, 0)),
        out_shape=jax.ShapeDtypeStruct((B, NCLS), jnp.float32),
    )(x, w1t, b1r, w2t, b2r, w3t, b3r)


def kernel(title_inputs, review_inputs, emb, W1, b1, W2, b2, W3, b3):
    combined = _bow(title_inputs.reshape(-1), review_inputs.reshape(-1), emb)
    return _mlp(combined, W1.T, b1.reshape(1, -1),
                W2.T, b2.reshape(1, -1), W3.T, b3.reshape(1, -1))


# speedup vs baseline: 1.1150x; 1.1150x over previous
import functools

import jax
import jax.numpy as jnp
from jax import lax
from jax.experimental import pallas as pl
from jax.experimental.pallas import tpu as pltpu
from jax.experimental.pallas import tpu_sc as plsc

B = 4096
EMB = 64
T_LEN = 20
R_LEN = 200
H1 = 256
H2 = 128
NCLS = 2

NUM_CORES = 2
NUM_SUBCORES = 16
NW = NUM_CORES * NUM_SUBCORES
BPW = B // NW
NPAIR = BPW // 2

RC0 = 104
RC1 = R_LEN - RC0


def _bow_body(title_hbm, review_hbm, emb_hbm, out_hbm,
              tidx_v, ridx_v,
              tbufA, rbuf0A, rbuf1A, tbufB, rbuf0B, rbuf1B,
              acc_v, semA, semB):
    wid = lax.axis_index("s") * NUM_CORES + lax.axis_index("c")
    base = wid * BPW

    pltpu.sync_copy(title_hbm.at[pl.ds(base * T_LEN, BPW * T_LEN)], tidx_v)
    pltpu.sync_copy(review_hbm.at[pl.ds(base * R_LEN, BPW * R_LEN)], ridx_v)

    def fire(b2, tbuf, rbuf0, rbuf1, sem):
        r0 = 2 * b2
        r1 = r0 + 1
        pltpu.async_copy(emb_hbm.at[tidx_v.at[pl.ds(b2 * (2 * T_LEN), 2 * T_LEN)]], tbuf, sem)
        pltpu.async_copy(emb_hbm.at[ridx_v.at[pl.ds(r0 * R_LEN, RC0)]], rbuf0.at[pl.ds(0, RC0)], sem)
        pltpu.async_copy(emb_hbm.at[ridx_v.at[pl.ds(r0 * R_LEN + RC0, RC1)]], rbuf0.at[pl.ds(RC0, RC1)], sem)
        pltpu.async_copy(emb_hbm.at[ridx_v.at[pl.ds(r1 * R_LEN, RC0)]], rbuf1.at[pl.ds(0, RC0)], sem)
        pltpu.async_copy(emb_hbm.at[ridx_v.at[pl.ds(r1 * R_LEN + RC0, RC1)]], rbuf1.at[pl.ds(RC0, RC1)], sem)

    def drain(tbuf, rbuf0, rbuf1, sem):
        pltpu.make_async_copy(emb_hbm.at[pl.ds(0, 2 * T_LEN)], tbuf, sem).wait()
        pltpu.make_async_copy(emb_hbm.at[pl.ds(0, R_LEN)], rbuf0, sem).wait()
        pltpu.make_async_copy(emb_hbm.at[pl.ds(0, R_LEN)], rbuf1, sem).wait()

    def process(b2, tbuf, rbuf0, rbuf1):
        r0 = 2 * b2
        r1 = r0 + 1
        zero = jnp.zeros((16,), jnp.float32)

        def tsum(t, accs):
            return tuple(a + tbuf[t, pl.ds(16 * c, 16)] for c, a in enumerate(accs))

        ta0 = lax.fori_loop(0, T_LEN, tsum, (zero,) * 4, unroll=4)
        ta1 = lax.fori_loop(T_LEN, 2 * T_LEN, tsum, (zero,) * 4, unroll=4)

        def rsum0(t, accs):
            return tuple(a + rbuf0[t, pl.ds(16 * c, 16)] for c, a in enumerate(accs))

        def rsum1(t, accs):
            return tuple(a + rbuf1[t, pl.ds(16 * c, 16)] for c, a in enumerate(accs))

        ra0 = lax.fori_loop(0, R_LEN, rsum0, (zero,) * 4, unroll=8)
        ra1 = lax.fori_loop(0, R_LEN, rsum1, (zero,) * 4, unroll=8)

        for c in range(4):
            acc_v[r0, pl.ds(16 * c, 16)] = ta0[c]
            acc_v[r1, pl.ds(16 * c, 16)] = ta1[c]
            acc_v[r0, pl.ds(EMB + 16 * c, 16)] = ra0[c]
            acc_v[r1, pl.ds(EMB + 16 * c, 16)] = ra1[c]

    fire(0, tbufA, rbuf0A, rbuf1A, semA)
    fire(1, tbufB, rbuf0B, rbuf1B, semB)

    def body(j, carry):
        pA = 2 * j
        pB = pA + 1
        drain(tbufA, rbuf0A, rbuf1A, semA)
        process(pA, tbufA, rbuf0A, rbuf1A)
        fire(pA + 2, tbufA, rbuf0A, rbuf1A, semA)
        drain(tbufB, rbuf0B, rbuf1B, semB)
        process(pB, tbufB, rbuf0B, rbuf1B)
        fire(pB + 2, tbufB, rbuf0B, rbuf1B, semB)
        return carry

    lax.fori_loop(0, NPAIR // 2 - 1, body, 0)
    drain(tbufA, rbuf0A, rbuf1A, semA)
    process(NPAIR - 2, tbufA, rbuf0A, rbuf1A)
    drain(tbufB, rbuf0B, rbuf1B, semB)
    process(NPAIR - 1, tbufB, rbuf0B, rbuf1B)

    pltpu.sync_copy(acc_v, out_hbm.at[pl.ds(base, BPW)])


@jax.jit
def _bow(title_flat, review_flat, emb):
    mesh = plsc.VectorSubcoreMesh(core_axis_name="c", subcore_axis_name="s")
    return pl.kernel(
        _bow_body,
        out_type=jax.ShapeDtypeStruct((B, 2 * EMB), jnp.float32),
        mesh=mesh,
        scratch_types=[
            pltpu.VMEM((BPW * T_LEN,), jnp.int32),
            pltpu.VMEM((BPW * R_LEN,), jnp.int32),
            pltpu.VMEM((2 * T_LEN, EMB), jnp.float32),
            pltpu.VMEM((R_LEN, EMB), jnp.float32),
            pltpu.VMEM((R_LEN, EMB), jnp.float32),
            pltpu.VMEM((2 * T_LEN, EMB), jnp.float32),
            pltpu.VMEM((R_LEN, EMB), jnp.float32),
            pltpu.VMEM((R_LEN, EMB), jnp.float32),
            pltpu.VMEM((BPW, 2 * EMB), jnp.float32),
            pltpu.SemaphoreType.DMA,
            pltpu.SemaphoreType.DMA,
        ],
        compiler_params=pltpu.CompilerParams(use_tc_tiling_on_sc=False),
    )(title_flat, review_flat, emb)


def _mlp_body(x_ref, w1_ref, b1_ref, w2_ref, b2_ref, w3_ref, b3_ref, o_ref):
    x = x_ref[...]
    h = jnp.dot(x, w1_ref[...], preferred_element_type=jnp.float32) + b1_ref[...]
    h = jnp.maximum(h, 0.0)
    h = jnp.dot(h, w2_ref[...], preferred_element_type=jnp.float32) + b2_ref[...]
    h = jnp.maximum(h, 0.0)
    o_ref[...] = jnp.dot(h, w3_ref[...], preferred_element_type=jnp.float32) + b3_ref[...]


def _mlp(x, w1t, b1r, w2t, b2r, w3t, b3r):
    BLK = 512
    return pl.pallas_call(
        _mlp_body,
        grid=(B // BLK,),
        in_specs=[
            pl.BlockSpec((BLK, 2 * EMB), lambda i: (i, 0)),
            pl.BlockSpec((2 * EMB, H1), lambda i: (0, 0)),
            pl.BlockSpec((1, H1), lambda i: (0, 0)),
            pl.BlockSpec((H1, H2), lambda i: (0, 0)),
            pl.BlockSpec((1, H2), lambda i: (0, 0)),
            pl.BlockSpec((H2, NCLS), lambda i: (0, 0)),
            pl.BlockSpec((1, NCLS), lambda i: (0, 0)),
        ],
        out_specs=pl.BlockSpec((BLK, NCLS), lambda i: (i, 0)),
        out_shape=jax.ShapeDtypeStruct((B, NCLS), jnp.float32),
    )(x, w1t, b1r, w2t, b2r, w3t, b3r)


def kernel(title_inputs, review_inputs, emb, W1, b1, W2, b2, W3, b3):
    combined = _bow(title_inputs.reshape(-1), review_inputs.reshape(-1), emb)
    return _mlp(combined, W1.T, b1.reshape(1, -1),
                W2.T, b2.reshape(1, -1), W3.T, b3.reshape(1, -1))
